# Initial kernel scaffold; baseline (speedup 1.0000x reference)
import jax
import jax.numpy as jnp
from jax.experimental import pallas as pl


def kernel(x, table):
    raise NotImplementedError("write your pallas kernel here")


# speedup vs baseline: 2.4095x; 2.4095x over previous
import functools
import math

import jax
import jax.numpy as jnp
from jax import lax
from jax.experimental import pallas as pl
from jax.experimental.pallas import tpu as pltpu
from jax.experimental.pallas import tpu_sc as plsc

D_MODEL = 128
SCALE = math.sqrt(D_MODEL)
CHUNK = 128
LANES = 16


def _emb_kernel(n_total, n_per_w, table_hbm, idx_hbm, out_hbm,
                idx_v, buf, sem):
    nc = 2
    wid = lax.axis_index("s") * nc + lax.axis_index("c")
    base = wid * n_per_w
    pltpu.sync_copy(idx_hbm.at[pl.ds(base, n_per_w)], idx_v)

    n_chunks = n_per_w // CHUNK

    def chunk_body(j, _):
        pltpu.async_copy(
            table_hbm.at[idx_v.at[pl.ds(j * CHUNK, CHUNK)]], buf, sem
        ).wait()

        def row_body(r, _):
            for t in range(D_MODEL // LANES):
                sl = pl.ds(t * LANES, LANES)
                buf[r, sl] = buf[r, sl] * SCALE
            return 0

        lax.fori_loop(0, CHUNK, row_body, 0)
        pltpu.sync_copy(buf, out_hbm.at[pl.ds(base + j * CHUNK, CHUNK)])
        return 0

    lax.fori_loop(0, n_chunks, chunk_body, 0)


def kernel(x, table):
    n_total = x.shape[0] * x.shape[1]
    idx = x.reshape(-1).astype(jnp.int32)
    nw = 32
    n_per_w = n_total // nw
    mesh = plsc.VectorSubcoreMesh(core_axis_name="c", subcore_axis_name="s")
    k = pl.kernel(
        functools.partial(_emb_kernel, n_total, n_per_w),
        mesh=mesh,
        out_type=jax.ShapeDtypeStruct((n_total, D_MODEL), jnp.float32),
        scratch_types=[
            pltpu.VMEM((n_per_w,), jnp.int32),
            pltpu.VMEM((CHUNK, D_MODEL), jnp.float32),
            pltpu.SemaphoreType.DMA,
        ],
    )
    out = k(table, idx)
    return out.reshape(x.shape[0], x.shape[1], D_MODEL)


# speedup vs baseline: 2.9442x; 1.2219x over previous
import functools
import math

import jax
import jax.numpy as jnp
from jax import lax
from jax.experimental import pallas as pl
from jax.experimental.pallas import tpu as pltpu
from jax.experimental.pallas import tpu_sc as plsc

D_MODEL = 128
SCALE = math.sqrt(D_MODEL)
CHUNK = 128
LANES = 16
NBUF = 2


def _scale_chunk(src, dst):
    def row_body(r, _):
        for t in range(D_MODEL // LANES):
            sl = pl.ds(t * LANES, LANES)
            dst[r, sl] = src[r, sl] * SCALE
        return 0

    lax.fori_loop(0, CHUNK, row_body, 0)


def _emb_kernel(n_per_w, table_hbm, idx_hbm, out_hbm, idx_v,
                gbuf0, gbuf1, obuf0, obuf1,
                gsem0, gsem1, osem0, osem1):
    nc = 2
    wid = lax.axis_index("s") * nc + lax.axis_index("c")
    base = wid * n_per_w
    pltpu.sync_copy(idx_hbm.at[pl.ds(base, n_per_w)], idx_v)

    gbufs = (gbuf0, gbuf1)
    obufs = (obuf0, obuf1)
    gsems = (gsem0, gsem1)
    osems = (osem0, osem1)
    n_outer = n_per_w // (CHUNK * NBUF)

    def gather_chunk(j, b):
        return pltpu.make_async_copy(
            table_hbm.at[idx_v.at[pl.ds(j * CHUNK, CHUNK)]],
            gbufs[b], gsems[b])

    for b in range(NBUF):
        gather_chunk(b, b).start()

    def outer_body(g, _):
        for b in range(NBUF):
            j = g * NBUF + b
            pltpu.make_async_copy(
                table_hbm.at[idx_v.at[pl.ds(j * CHUNK, CHUNK)]],
                gbufs[b], gsems[b]).wait()

            @pl.when(g > 0)
            def _wait_prev_out():
                pltpu.make_async_copy(
                    obufs[b],
                    out_hbm.at[pl.ds(base + (j - NBUF) * CHUNK, CHUNK)],
                    osems[b]).wait()

            _scale_chunk(gbufs[b], obufs[b])

            pltpu.make_async_copy(
                obufs[b],
                out_hbm.at[pl.ds(base + j * CHUNK, CHUNK)],
                osems[b]).start()

            @pl.when(g < n_outer - 1)
            def _start_next_gather():
                pltpu.make_async_copy(
                    table_hbm.at[idx_v.at[pl.ds((j + NBUF) * CHUNK, CHUNK)]],
                    gbufs[b], gsems[b]).start()
        return 0

    lax.fori_loop(0, n_outer, outer_body, 0)

    for b in range(NBUF):
        j = (n_outer - 1) * NBUF + b
        pltpu.make_async_copy(
            obufs[b],
            out_hbm.at[pl.ds(base + j * CHUNK, CHUNK)],
            osems[b]).wait()


def kernel(x, table):
    n_total = x.shape[0] * x.shape[1]
    idx = x.reshape(-1).astype(jnp.int32)
    nw = 32
    n_per_w = n_total // nw
    mesh = plsc.VectorSubcoreMesh(core_axis_name="c", subcore_axis_name="s")
    k = pl.kernel(
        functools.partial(_emb_kernel, n_per_w),
        mesh=mesh,
        out_type=jax.ShapeDtypeStruct((n_total, D_MODEL), jnp.float32),
        scratch_types=[
            pltpu.VMEM((n_per_w,), jnp.int32),
            pltpu.VMEM((CHUNK, D_MODEL), jnp.float32),
            pltpu.VMEM((CHUNK, D_MODEL), jnp.float32),
            pltpu.VMEM((CHUNK, D_MODEL), jnp.float32),
            pltpu.VMEM((CHUNK, D_MODEL), jnp.float32),
            pltpu.SemaphoreType.DMA,
            pltpu.SemaphoreType.DMA,
            pltpu.SemaphoreType.DMA,
            pltpu.SemaphoreType.DMA,
        ],
    )
    out = k(table, idx)
    return out.reshape(x.shape[0], x.shape[1], D_MODEL)


# speedup vs baseline: 9.1194x; 3.0974x over previous
import functools
import math

import jax
import jax.numpy as jnp
from jax import lax
from jax.experimental import pallas as pl
from jax.experimental.pallas import tpu as pltpu
from jax.experimental.pallas import tpu_sc as plsc

D_MODEL = 128
SCALE = math.sqrt(D_MODEL)
CHUNK = 128
LANES = 16
NBUF = 2


def _scale_chunk(src, dst):
    def row_body(r, _):
        for t in range(D_MODEL // LANES):
            sl = pl.ds(t * LANES, LANES)
            dst[r, sl] = src[r, sl] * SCALE
        return 0

    lax.fori_loop(0, CHUNK, row_body, 0)


def _emb_kernel(n_per_w, table_hbm, idx_hbm, out_hbm, idx_v,
                gbuf0, gbuf1, obuf0, obuf1,
                gsem0, gsem1, osem0, osem1):
    nc = 2
    wid = lax.axis_index("s") * nc + lax.axis_index("c")
    base = wid * n_per_w
    pltpu.sync_copy(idx_hbm.at[pl.ds(base, n_per_w)], idx_v)

    gbufs = (gbuf0, gbuf1)
    obufs = (obuf0, obuf1)
    gsems = (gsem0, gsem1)
    osems = (osem0, osem1)
    n_outer = n_per_w // (CHUNK * NBUF)

    def gather_chunk(j, b):
        return pltpu.make_async_copy(
            table_hbm.at[idx_v.at[pl.ds(j * CHUNK, CHUNK)]],
            gbufs[b], gsems[b])

    for b in range(NBUF):
        gather_chunk(b, b).start()

    def outer_body(g, _):
        for b in range(NBUF):
            j = g * NBUF + b
            pltpu.make_async_copy(
                table_hbm.at[idx_v.at[pl.ds(j * CHUNK, CHUNK)]],
                gbufs[b], gsems[b]).wait()

            @pl.when(g > 0)
            def _wait_prev_out():
                pltpu.make_async_copy(
                    obufs[b],
                    out_hbm.at[pl.ds(base + (j - NBUF) * CHUNK, CHUNK)],
                    osems[b]).wait()

            _scale_chunk(gbufs[b], obufs[b])

            pltpu.make_async_copy(
                obufs[b],
                out_hbm.at[pl.ds(base + j * CHUNK, CHUNK)],
                osems[b]).start()

            @pl.when(g < n_outer - 1)
            def _start_next_gather():
                pltpu.make_async_copy(
                    table_hbm.at[idx_v.at[pl.ds((j + NBUF) * CHUNK, CHUNK)]],
                    gbufs[b], gsems[b]).start()
        return 0

    lax.fori_loop(0, n_outer, outer_body, 0)

    for b in range(NBUF):
        j = (n_outer - 1) * NBUF + b
        pltpu.make_async_copy(
            obufs[b],
            out_hbm.at[pl.ds(base + j * CHUNK, CHUNK)],
            osems[b]).wait()


def kernel(x, table):
    n_total = x.shape[0] * x.shape[1]
    idx = x.T.reshape(-1).astype(jnp.int32)
    nw = 32
    n_per_w = n_total // nw
    mesh = plsc.VectorSubcoreMesh(core_axis_name="c", subcore_axis_name="s")
    k = pl.kernel(
        functools.partial(_emb_kernel, n_per_w),
        mesh=mesh,
        out_type=jax.ShapeDtypeStruct((n_total, D_MODEL), jnp.float32),
        scratch_types=[
            pltpu.VMEM((n_per_w,), jnp.int32),
            pltpu.VMEM((CHUNK, D_MODEL), jnp.float32),
            pltpu.VMEM((CHUNK, D_MODEL), jnp.float32),
            pltpu.VMEM((CHUNK, D_MODEL), jnp.float32),
            pltpu.VMEM((CHUNK, D_MODEL), jnp.float32),
            pltpu.SemaphoreType.DMA,
            pltpu.SemaphoreType.DMA,
            pltpu.SemaphoreType.DMA,
            pltpu.SemaphoreType.DMA,
        ],
    )
    out = k(table, idx)
    return out.reshape(x.shape[1], x.shape[0], D_MODEL).transpose(1, 0, 2)


# speedup vs baseline: 9.2060x; 1.0095x over previous
import functools
import math

import jax
import jax.numpy as jnp
from jax import lax
from jax.experimental import pallas as pl
from jax.experimental.pallas import tpu as pltpu
from jax.experimental.pallas import tpu_sc as plsc

D_MODEL = 128
SCALE = math.sqrt(D_MODEL)
CHUNK = 64
LANES = 16
NBUF = 4


def _scale_chunk(src, dst):
    def row_body(r, _):
        for t in range(D_MODEL // LANES):
            sl = pl.ds(t * LANES, LANES)
            dst[r, sl] = src[r, sl] * SCALE
        return 0

    lax.fori_loop(0, CHUNK, row_body, 0)


def _emb_kernel(n_per_w, table_hbm, idx_hbm, out_hbm, idx_v, *scratch):
    gbufs = scratch[:NBUF]
    obufs = scratch[NBUF:2 * NBUF]
    gsems = scratch[2 * NBUF:3 * NBUF]
    osems = scratch[3 * NBUF:4 * NBUF]

    nc = 2
    wid = lax.axis_index("s") * nc + lax.axis_index("c")
    base = wid * n_per_w
    pltpu.sync_copy(idx_hbm.at[pl.ds(base, n_per_w)], idx_v)

    n_outer = n_per_w // (CHUNK * NBUF)

    for b in range(NBUF):
        pltpu.make_async_copy(
            table_hbm.at[idx_v.at[pl.ds(b * CHUNK, CHUNK)]],
            gbufs[b], gsems[b]).start()

    def outer_body(g, _):
        for b in range(NBUF):
            j = g * NBUF + b
            pltpu.make_async_copy(
                table_hbm.at[idx_v.at[pl.ds(j * CHUNK, CHUNK)]],
                gbufs[b], gsems[b]).wait()

            @pl.when(g > 0)
            def _wait_prev_out():
                pltpu.make_async_copy(
                    obufs[b],
                    out_hbm.at[pl.ds(base + (j - NBUF) * CHUNK, CHUNK)],
                    osems[b]).wait()

            _scale_chunk(gbufs[b], obufs[b])

            pltpu.make_async_copy(
                obufs[b],
                out_hbm.at[pl.ds(base + j * CHUNK, CHUNK)],
                osems[b]).start()

            @pl.when(g < n_outer - 1)
            def _start_next_gather():
                pltpu.make_async_copy(
                    table_hbm.at[idx_v.at[pl.ds((j + NBUF) * CHUNK, CHUNK)]],
                    gbufs[b], gsems[b]).start()
        return 0

    lax.fori_loop(0, n_outer, outer_body, 0)

    for b in range(NBUF):
        j = (n_outer - 1) * NBUF + b
        pltpu.make_async_copy(
            obufs[b],
            out_hbm.at[pl.ds(base + j * CHUNK, CHUNK)],
            osems[b]).wait()


def kernel(x, table):
    n_total = x.shape[0] * x.shape[1]
    idx = x.T.reshape(-1).astype(jnp.int32)
    nw = 32
    n_per_w = n_total // nw
    mesh = plsc.VectorSubcoreMesh(core_axis_name="c", subcore_axis_name="s")
    k = pl.kernel(
        functools.partial(_emb_kernel, n_per_w),
        mesh=mesh,
        out_type=jax.ShapeDtypeStruct((n_total, D_MODEL), jnp.float32),
        scratch_types=(
            [pltpu.VMEM((n_per_w,), jnp.int32)]
            + [pltpu.VMEM((CHUNK, D_MODEL), jnp.float32)] * (2 * NBUF)
            + [pltpu.SemaphoreType.DMA] * (2 * NBUF)
        ),
    )
    out = k(table, idx)
    return out.reshape(x.shape[1], x.shape[0], D_MODEL).transpose(1, 0, 2)
